# baseline (speedup 1.0000x reference)
import functools

import jax
import jax.numpy as jnp
from jax import lax
from jax.experimental import pallas as pl
from jax.experimental.pallas import tpu as pltpu
from jax.experimental.pallas import tpu_sc as plsc

B = 4096
S = 50
P = 20
T = P + S
D = 128

_info = plsc.get_sparse_core_info()
NC, NS = _info.num_cores, _info.num_subcores
NW = NC * NS
RW = B // NW
NBUF = 2


def _make_kernel():
    mesh = plsc.VectorSubcoreMesh(core_axis_name="c", subcore_axis_name="s")

    @functools.partial(
        pl.kernel,
        mesh=mesh,
        compiler_params=pltpu.CompilerParams(use_tc_tiling_on_sc=True),
        out_type=jax.ShapeDtypeStruct((T * B, D), jnp.float32),
        scratch_types=[
            pltpu.VMEM((T, RW), jnp.int32),
            pltpu.VMEM((RW, D), jnp.float32),
            pltpu.VMEM((RW, D), jnp.float32),
            pltpu.SemaphoreType.DMA,
            pltpu.SemaphoreType.DMA,
            pltpu.SemaphoreType.DMA,
            pltpu.SemaphoreType.DMA,
        ],
    )
    def k(ids_hbm, table_hbm, prompt_hbm, out_hbm,
          idx_v, buf0, buf1, g0, g1, w0, w1):
        bufs = (buf0, buf1)
        gsems = (g0, g1)
        wsems = (w0, w1)
        wid = lax.axis_index("s") * NC + lax.axis_index("c")
        base = wid * RW

        for t in range(P):
            for c in range(RW // 16):
                idx_v[t, pl.ds(c * 16, 16)] = jnp.full((16,), t, jnp.int32)
        pltpu.sync_copy(ids_hbm.at[:, pl.ds(base, RW)],
                        idx_v.at[pl.ds(P, S)])

        def drain_gather(s):
            pltpu.make_async_copy(
                table_hbm.at[pl.ds(0, RW)], bufs[s], gsems[s]).wait()

        def fire_write(t, s):
            pltpu.async_copy(
                bufs[s], out_hbm.at[pl.ds(t * B + base, RW)], wsems[s])

        def drain_write(s):
            pltpu.make_async_copy(
                bufs[s], out_hbm.at[pl.ds(0, RW)], wsems[s]).wait()

        def run_phase(src_hbm, t0, nsteps):
            def fire_gather(t, s):
                pltpu.async_copy(
                    src_hbm.at[idx_v.at[t]], bufs[s], gsems[s])

            for s in range(NBUF):
                fire_gather(t0 + s, s)

            def outer(c, carry):
                tt = t0 + c * NBUF
                for s in range(NBUF):
                    drain_gather(s)
                    fire_write(tt + s, s)
                for s in range(NBUF):
                    drain_write(s)
                    fire_gather(tt + NBUF + s, s)
                return carry

            lax.fori_loop(0, nsteps // NBUF - 1, outer, 0)

            for s in range(NBUF):
                drain_gather(s)
                fire_write(t0 + nsteps - NBUF + s, s)
            for s in range(NBUF):
                drain_write(s)

        run_phase(prompt_hbm, 0, P)
        run_phase(table_hbm, P, S)

    return k


_kernel = _make_kernel()


def kernel(input_ids, embed_table, prompt_weight):
    ids_t = input_ids.astype(jnp.int32).T
    out = _kernel(ids_t, embed_table, prompt_weight)
    return out.reshape(T, B, D).transpose(1, 0, 2)


# speedup vs baseline: 5.7707x; 5.7707x over previous
import functools

import jax
import jax.numpy as jnp
from jax import lax
from jax.experimental import pallas as pl
from jax.experimental.pallas import tpu as pltpu
from jax.experimental.pallas import tpu_sc as plsc

B = 4096
S = 50
P = 20
T = P + S
D = 128

_info = plsc.get_sparse_core_info()
NC, NS = _info.num_cores, _info.num_subcores
NW = NC * NS
RW = B // NW
NBUF = 2


def _make_kernel():
    mesh = plsc.VectorSubcoreMesh(core_axis_name="c", subcore_axis_name="s")

    @functools.partial(
        pl.kernel,
        mesh=mesh,
        compiler_params=pltpu.CompilerParams(use_tc_tiling_on_sc=True),
        out_type=jax.ShapeDtypeStruct((T * B, D), jnp.float32),
        scratch_types=[
            pltpu.VMEM((T, RW), jnp.int32),
            pltpu.VMEM((RW, D), jnp.float32),
            pltpu.VMEM((RW, D), jnp.float32),
            pltpu.SemaphoreType.DMA,
            pltpu.SemaphoreType.DMA,
            pltpu.SemaphoreType.DMA,
            pltpu.SemaphoreType.DMA,
        ],
    )
    def k(ids_hbm, table_hbm, prompt_hbm, out_hbm,
          idx_v, buf0, buf1, g0, g1, w0, w1):
        bufs = (buf0, buf1)
        gsems = (g0, g1)
        wsems = (w0, w1)
        wid = lax.axis_index("s") * NC + lax.axis_index("c")
        base = wid * RW

        for t in range(P):
            for c in range(RW // 16):
                idx_v[t, pl.ds(c * 16, 16)] = (
                    t + P * c * 16 + P * lax.iota(jnp.int32, 16))
        pltpu.sync_copy(ids_hbm.at[:, pl.ds(base, RW)],
                        idx_v.at[pl.ds(P, S)])

        def drain_gather(s):
            pltpu.make_async_copy(
                table_hbm.at[pl.ds(0, RW)], bufs[s], gsems[s]).wait()

        def fire_write(t, s):
            pltpu.async_copy(
                bufs[s], out_hbm.at[pl.ds(t * B + base, RW)], wsems[s])

        def drain_write(s):
            pltpu.make_async_copy(
                bufs[s], out_hbm.at[pl.ds(0, RW)], wsems[s]).wait()

        def run_phase(src_hbm, t0, nsteps):
            def fire_gather(t, s):
                pltpu.async_copy(
                    src_hbm.at[idx_v.at[t]], bufs[s], gsems[s])

            for s in range(NBUF):
                fire_gather(t0 + s, s)

            def outer(c, carry):
                tt = t0 + c * NBUF
                for s in range(NBUF):
                    drain_gather(s)
                    fire_write(tt + s, s)
                for s in range(NBUF):
                    drain_write(s)
                    fire_gather(tt + NBUF + s, s)
                return carry

            lax.fori_loop(0, nsteps // NBUF - 1, outer, 0)

            for s in range(NBUF):
                drain_gather(s)
                fire_write(t0 + nsteps - NBUF + s, s)
            for s in range(NBUF):
                drain_write(s)

        run_phase(prompt_hbm, 0, P)
        run_phase(table_hbm, P, S)

    return k


_kernel = _make_kernel()


def kernel(input_ids, embed_table, prompt_weight):
    ids_t = input_ids.astype(jnp.int32).T
    prompt_rep = jnp.tile(prompt_weight, (RW, 1))
    out = _kernel(ids_t, embed_table, prompt_rep)
    return out.reshape(T, B, D).transpose(1, 0, 2)


# speedup vs baseline: 6.4645x; 1.1202x over previous
import functools

import jax
import jax.numpy as jnp
from jax import lax
from jax.experimental import pallas as pl
from jax.experimental.pallas import tpu as pltpu
from jax.experimental.pallas import tpu_sc as plsc

B = 4096
S = 50
P = 20
T = P + S
D = 128

_info = plsc.get_sparse_core_info()
NC, NS = _info.num_cores, _info.num_subcores
NW = NC * NS
RW = B // NW
NBUF = 5


def _make_kernel():
    mesh = plsc.VectorSubcoreMesh(core_axis_name="c", subcore_axis_name="s")

    @functools.partial(
        pl.kernel,
        mesh=mesh,
        compiler_params=pltpu.CompilerParams(use_tc_tiling_on_sc=True),
        out_type=jax.ShapeDtypeStruct((T * B, D), jnp.float32),
        scratch_types=[
            pltpu.VMEM((T, RW), jnp.int32),
            *([pltpu.VMEM((RW, D), jnp.float32)] * 5),
            *([pltpu.SemaphoreType.DMA] * 10),
        ],
    )
    def k(ids_hbm, table_hbm, prompt_hbm, out_hbm,
          idx_v, b0, b1, b2, b3, b4,
          g0, g1, g2, g3, g4, w0, w1, w2, w3, w4):
        bufs = (b0, b1, b2, b3, b4)
        gsems = (g0, g1, g2, g3, g4)
        wsems = (w0, w1, w2, w3, w4)
        wid = lax.axis_index("s") * NC + lax.axis_index("c")
        base = wid * RW

        for t in range(P):
            for c in range(RW // 16):
                idx_v[t, pl.ds(c * 16, 16)] = (
                    t + P * c * 16 + P * lax.iota(jnp.int32, 16))
        pltpu.sync_copy(ids_hbm.at[:, pl.ds(base, RW)],
                        idx_v.at[pl.ds(P, S)])

        def drain_gather(s):
            pltpu.make_async_copy(
                table_hbm.at[pl.ds(0, RW)], bufs[s], gsems[s]).wait()

        def fire_write(t, s):
            pltpu.async_copy(
                bufs[s], out_hbm.at[pl.ds(t * B + base, RW)], wsems[s])

        def drain_write(s):
            pltpu.make_async_copy(
                bufs[s], out_hbm.at[pl.ds(0, RW)], wsems[s]).wait()

        def run_phase(src_hbm, t0, nsteps):
            def fire_gather(t, s):
                pltpu.async_copy(
                    src_hbm.at[idx_v.at[t]], bufs[s], gsems[s])

            for s in range(NBUF):
                fire_gather(t0 + s, s)

            def outer(c, carry):
                tt = t0 + c * NBUF
                for s in range(NBUF):
                    drain_gather(s)
                    fire_write(tt + s, s)
                for s in range(NBUF):
                    drain_write(s)
                    fire_gather(tt + NBUF + s, s)
                return carry

            lax.fori_loop(0, nsteps // NBUF - 1, outer, 0)

            for s in range(NBUF):
                drain_gather(s)
                fire_write(t0 + nsteps - NBUF + s, s)
            for s in range(NBUF):
                drain_write(s)

        run_phase(prompt_hbm, 0, P)
        run_phase(table_hbm, P, S)

    return k


_kernel = _make_kernel()


def kernel(input_ids, embed_table, prompt_weight):
    ids_t = input_ids.astype(jnp.int32).T
    prompt_rep = jnp.tile(prompt_weight, (RW, 1))
    out = _kernel(ids_t, embed_table, prompt_rep)
    return out.reshape(T, B, D).transpose(1, 0, 2)


# speedup vs baseline: 6.8745x; 1.0634x over previous
import functools

import jax
import jax.numpy as jnp
from jax import lax
from jax.experimental import pallas as pl
from jax.experimental.pallas import tpu as pltpu
from jax.experimental.pallas import tpu_sc as plsc

B = 4096
S = 50
P = 20
T = P + S
D = 128

_info = plsc.get_sparse_core_info()
NC, NS = _info.num_cores, _info.num_subcores
NW = NC * NS
RW = B // NW
NBUF = 5
G = 16


def _make_kernel():
    mesh = plsc.VectorSubcoreMesh(core_axis_name="c", subcore_axis_name="s")

    @functools.partial(
        pl.kernel,
        mesh=mesh,
        compiler_params=pltpu.CompilerParams(use_tc_tiling_on_sc=True),
        out_type=jax.ShapeDtypeStruct((T, B, D), jnp.float32),
        scratch_types=[
            pltpu.VMEM((S, RW), jnp.int32),
            pltpu.VMEM((P, G, D), jnp.float32),
            *([pltpu.VMEM((RW, D), jnp.float32)] * 5),
            *([pltpu.SemaphoreType.DMA] * 11),
        ],
    )
    def k(ids_hbm, table_hbm, prompt_hbm, out_hbm,
          idx_v, pbuf, b0, b1, b2, b3, b4,
          g0, g1, g2, g3, g4, w0, w1, w2, w3, w4, psem):
        bufs = (b0, b1, b2, b3, b4)
        gsems = (g0, g1, g2, g3, g4)
        wsems = (w0, w1, w2, w3, w4)
        wid = lax.axis_index("s") * NC + lax.axis_index("c")
        base = wid * RW

        pltpu.sync_copy(ids_hbm.at[:, pl.ds(base, RW)], idx_v)

        for j in range(G):
            pltpu.async_copy(prompt_hbm, pbuf.at[:, pl.ds(j, 1)], psem)
        for j in range(G):
            pltpu.make_async_copy(
                prompt_hbm, pbuf.at[:, pl.ds(0, 1)], psem).wait()
        for g in range(RW // G):
            pltpu.async_copy(
                pbuf, out_hbm.at[pl.ds(0, P), pl.ds(base + g * G, G)], psem)

        def drain_gather(s):
            pltpu.make_async_copy(
                table_hbm.at[pl.ds(0, RW)], bufs[s], gsems[s]).wait()

        def fire_write(t, s):
            pltpu.async_copy(
                bufs[s], out_hbm.at[t, pl.ds(base, RW)], wsems[s])

        def drain_write(s):
            pltpu.make_async_copy(
                bufs[s], out_hbm.at[0, pl.ds(0, RW)], wsems[s]).wait()

        def fire_gather(t, s):
            pltpu.async_copy(
                table_hbm.at[idx_v.at[t - P]], bufs[s], gsems[s])

        for s in range(NBUF):
            fire_gather(P + s, s)

        def outer(c, carry):
            tt = P + c * NBUF
            for s in range(NBUF):
                drain_gather(s)
                fire_write(tt + s, s)
            for s in range(NBUF):
                drain_write(s)
                fire_gather(tt + NBUF + s, s)
            return carry

        lax.fori_loop(0, S // NBUF - 1, outer, 0)

        for s in range(NBUF):
            drain_gather(s)
            fire_write(T - NBUF + s, s)
        for s in range(NBUF):
            drain_write(s)

        for g in range(RW // G):
            pltpu.make_async_copy(
                pbuf, out_hbm.at[pl.ds(0, P), pl.ds(0, G)], psem).wait()

    return k


_kernel = _make_kernel()


def kernel(input_ids, embed_table, prompt_weight):
    ids_t = input_ids.astype(jnp.int32).T
    out = _kernel(ids_t, embed_table, prompt_weight.reshape(P, 1, D))
    return out.transpose(1, 0, 2)


# speedup vs baseline: 6.9085x; 1.0049x over previous
import functools

import jax
import jax.numpy as jnp
from jax import lax
from jax.experimental import pallas as pl
from jax.experimental.pallas import tpu as pltpu
from jax.experimental.pallas import tpu_sc as plsc

B = 4096
S = 50
P = 20
T = P + S
D = 128

_info = plsc.get_sparse_core_info()
NC, NS = _info.num_cores, _info.num_subcores
NW = NC * NS
RW = B // NW
NBUF = 10
HC = 64
G = 16


def _make_kernel():
    mesh = plsc.VectorSubcoreMesh(core_axis_name="c", subcore_axis_name="s")

    @functools.partial(
        pl.kernel,
        mesh=mesh,
        compiler_params=pltpu.CompilerParams(use_tc_tiling_on_sc=True),
        out_type=jax.ShapeDtypeStruct((T, B, D), jnp.float32),
        scratch_types=[
            pltpu.VMEM((S, RW), jnp.int32),
            pltpu.VMEM((P, G, D), jnp.float32),
            *([pltpu.VMEM((HC, D), jnp.float32)] * 10),
            *([pltpu.SemaphoreType.DMA] * 21),
        ],
    )
    def k(ids_hbm, table_hbm, prompt_hbm, out_hbm,
          idx_v, pbuf, b0, b1, b2, b3, b4, b5, b6, b7, b8, b9,
          g0, g1, g2, g3, g4, g5, g6, g7, g8, g9,
          w0, w1, w2, w3, w4, w5, w6, w7, w8, w9, psem):
        bufs = (b0, b1, b2, b3, b4, b5, b6, b7, b8, b9)
        gsems = (g0, g1, g2, g3, g4, g5, g6, g7, g8, g9)
        wsems = (w0, w1, w2, w3, w4, w5, w6, w7, w8, w9)
        wid = lax.axis_index("s") * NC + lax.axis_index("c")
        base = wid * RW

        pltpu.sync_copy(ids_hbm.at[:, pl.ds(base, RW)], idx_v)

        for j in range(G):
            pltpu.async_copy(prompt_hbm, pbuf.at[:, j], psem)
        for j in range(G):
            pltpu.make_async_copy(
                prompt_hbm, pbuf.at[:, 0], psem).wait()
        for g in range(RW // G):
            pltpu.async_copy(
                pbuf, out_hbm.at[pl.ds(0, P), pl.ds(base + g * G, G)], psem)

        def drain_gather(s):
            pltpu.make_async_copy(
                table_hbm.at[pl.ds(0, HC)], bufs[s], gsems[s]).wait()

        def fire_write(tk, h, s):
            pltpu.async_copy(
                bufs[s], out_hbm.at[P + tk, pl.ds(base + h * HC, HC)],
                wsems[s])

        def drain_write(s):
            pltpu.make_async_copy(
                bufs[s], out_hbm.at[0, pl.ds(0, HC)], wsems[s]).wait()

        def fire_gather(tk, h, s):
            pltpu.async_copy(
                table_hbm.at[idx_v.at[tk, pl.ds(h * HC, HC)]],
                bufs[s], gsems[s])

        NST = 2 * S
        for s in range(NBUF):
            fire_gather(s // 2, s % 2, s)

        def outer(c, carry):
            base_tk = c * (NBUF // 2)
            for s in range(NBUF):
                drain_gather(s)
                fire_write(base_tk + s // 2, s % 2, s)
            for s in range(NBUF):
                drain_write(s)
                fire_gather(base_tk + NBUF // 2 + s // 2, s % 2, s)
            return carry

        lax.fori_loop(0, NST // NBUF - 1, outer, 0)

        last_tk = S - NBUF // 2
        for s in range(NBUF):
            drain_gather(s)
            fire_write(last_tk + s // 2, s % 2, s)
        for s in range(NBUF):
            drain_write(s)

        for g in range(RW // G):
            pltpu.make_async_copy(
                pbuf, out_hbm.at[pl.ds(0, P), pl.ds(0, G)], psem).wait()

    return k


_kernel = _make_kernel()


def kernel(input_ids, embed_table, prompt_weight):
    ids_t = input_ids.astype(jnp.int32).T
    out = _kernel(ids_t, embed_table, prompt_weight)
    return out.transpose(1, 0, 2)
